# baseline (speedup 1.0000x reference)
import functools

import jax
import jax.numpy as jnp
from jax import lax
from jax.experimental import pallas as pl
from jax.experimental.pallas import tpu as pltpu
from jax.experimental.pallas import tpu_sc as plsc

B = 16384
V = 1000000
NF = 26
ND = 13
L = 16
NW = 32
BW = B // NW
CH = 128
NC_CHUNK = BW // CH
NG = BW // L


def _body(idx_hbm, dns_hbm, w_hbm, b_hbm, tab_hbm, out_hbm,
          idx_v, dns_v, w_v, b_v, gbuf, acc_v, sem):
    wid = lax.axis_index("s") * 2 + lax.axis_index("c")
    base = wid * BW

    pltpu.sync_copy(idx_hbm.at[wid], idx_v)
    pltpu.sync_copy(dns_hbm.at[wid], dns_v)
    pltpu.sync_copy(w_hbm, w_v)
    pltpu.sync_copy(b_hbm, b_v)

    waits = []
    for i in range(NF):
        for c in range(NC_CHUNK):
            waits.append(
                pltpu.async_copy(tab_hbm.at[idx_v.at[i, c]], gbuf.at[i, c], sem))
    for w in waits:
        w.wait()

    for g in range(NG):
        c, r = g // (CH // L), (g % (CH // L)) * L
        v = b_v[...]
        for d in range(ND):
            v = v + dns_v[d, pl.ds(g * L, L)] * w_v[d]
        for i in range(NF):
            v = v + gbuf[i, c, pl.ds(r, L)]
        acc_v[pl.ds(g * L, L)] = v

    pltpu.sync_copy(acc_v, out_hbm.at[pl.ds(base, BW)])


@jax.jit
def _run(idx_r, dns_r, w16, b16, tab):
    mesh = plsc.VectorSubcoreMesh(core_axis_name="c", subcore_axis_name="s")
    kfn = pl.kernel(
        _body,
        out_type=jax.ShapeDtypeStruct((B,), jnp.float32),
        mesh=mesh,
        scratch_types=[
            pltpu.VMEM((NF, NC_CHUNK, CH), jnp.int32),
            pltpu.VMEM((ND, BW), jnp.float32),
            pltpu.VMEM((ND, L), jnp.float32),
            pltpu.VMEM((L,), jnp.float32),
            pltpu.VMEM((NF, NC_CHUNK, CH), jnp.float32),
            pltpu.VMEM((BW,), jnp.float32),
            pltpu.SemaphoreType.DMA,
        ],
    )
    return kfn(idx_r, dns_r, w16, b16, tab)


def kernel(*args):
    cates = args[:NF]
    embs = args[NF:2 * NF]
    dense_feats, W, b = args[2 * NF:]

    idx = jnp.stack([c.reshape(B) for c in cates])
    idx = idx + (jnp.arange(NF, dtype=jnp.int32) * V)[:, None]
    idx_r = idx.reshape(NF, NW, NC_CHUNK, CH).transpose(1, 0, 2, 3)
    dns_r = dense_feats.T.reshape(ND, NW, BW).transpose(1, 0, 2)
    w16 = jnp.broadcast_to(W.reshape(ND, 1), (ND, L))
    b16 = jnp.broadcast_to(b.reshape(1), (L,))
    tab = jnp.concatenate([e.reshape(V) for e in embs])

    out = _run(idx_r, dns_r, w16, b16, tab)
    return out.reshape(B, 1)


# speedup vs baseline: 1.7086x; 1.7086x over previous
import functools

import jax
import jax.numpy as jnp
from jax import lax
from jax.experimental import pallas as pl
from jax.experimental.pallas import tpu as pltpu
from jax.experimental.pallas import tpu_sc as plsc

B = 16384
V = 1000000
NF = 26
ND = 13
L = 16
NW = 32
BW = B // NW
CH = 128
NC_CHUNK = BW // CH
NG = BW // L


def _body(idx_hbm, dns_hbm, w_hbm, b_hbm, *rest):
    embs = rest[:NF]
    out_hbm = rest[NF]
    idx_v, dns_v, w_v, b_v, gbuf, acc_v, sem = rest[NF + 1:]

    wid = lax.axis_index("s") * 2 + lax.axis_index("c")
    base = wid * BW

    pltpu.sync_copy(idx_hbm.at[wid], idx_v)
    pltpu.sync_copy(dns_hbm.at[wid], dns_v)
    pltpu.sync_copy(w_hbm, w_v)
    pltpu.sync_copy(b_hbm, b_v)

    waits = []
    for i in range(NF):
        for c in range(NC_CHUNK):
            waits.append(
                pltpu.async_copy(embs[i].at[idx_v.at[i, c]], gbuf.at[i, c], sem))
    for w in waits:
        w.wait()

    for g in range(NG):
        c, r = g // (CH // L), (g % (CH // L)) * L
        v = b_v[...]
        for d in range(ND):
            v = v + dns_v[d, pl.ds(g * L, L)] * w_v[d]
        for i in range(NF):
            v = v + gbuf[i, c, pl.ds(r, L)]
        acc_v[pl.ds(g * L, L)] = v

    pltpu.sync_copy(acc_v, out_hbm.at[pl.ds(base, BW)])


@jax.jit
def _run(idx_r, dns_r, w16, b16, *embs):
    mesh = plsc.VectorSubcoreMesh(core_axis_name="c", subcore_axis_name="s")
    kfn = pl.kernel(
        _body,
        out_type=jax.ShapeDtypeStruct((B,), jnp.float32),
        mesh=mesh,
        scratch_types=[
            pltpu.VMEM((NF, NC_CHUNK, CH), jnp.int32),
            pltpu.VMEM((ND, BW), jnp.float32),
            pltpu.VMEM((ND, L), jnp.float32),
            pltpu.VMEM((L,), jnp.float32),
            pltpu.VMEM((NF, NC_CHUNK, CH), jnp.float32),
            pltpu.VMEM((BW,), jnp.float32),
            pltpu.SemaphoreType.DMA,
        ],
    )
    return kfn(idx_r, dns_r, w16, b16, *embs)


def kernel(*args):
    cates = args[:NF]
    embs = args[NF:2 * NF]
    dense_feats, W, b = args[2 * NF:]

    idx = jnp.stack([c.reshape(B) for c in cates])
    idx_r = idx.reshape(NF, NW, NC_CHUNK, CH).transpose(1, 0, 2, 3)
    dns_r = dense_feats.T.reshape(ND, NW, BW).transpose(1, 0, 2)
    w16 = jnp.broadcast_to(W.reshape(ND, 1), (ND, L))
    b16 = jnp.broadcast_to(b.reshape(1), (L,))
    flat_embs = [e.reshape(V) for e in embs]

    out = _run(idx_r, dns_r, w16, b16, *flat_embs)
    return out.reshape(B, 1)
